# baseline (speedup 1.0000x reference)
import functools

import jax
import jax.numpy as jnp
from jax import lax
from jax.experimental import pallas as pl
from jax.experimental.pallas import tpu as pltpu
from jax.experimental.pallas import tpu_sc as plsc

NC = 2
NS = 16
L = 16
NW = NC * NS
_NSTAT = 5
_PACK = 4
_WORDS = 32

_SQRT2 = 1.4142135623730951
_LN2 = 0.6931471805599453


def _vlog(x):
    bits = lax.bitcast_convert_type(x, jnp.int32)
    e = lax.shift_right_logical(bits, 23) - 127
    m_bits = (bits & jnp.int32(0x7FFFFF)) | jnp.int32(0x3F800000)
    m = lax.bitcast_convert_type(m_bits, jnp.float32)
    big = m > _SQRT2
    m = jnp.where(big, 0.5 * m, m)
    e = e + jnp.where(big, 1, 0)
    ef = e.astype(jnp.float32)
    t = (m - 1.0) / (m + 1.0)
    t2 = t * t
    p = jnp.float32(1.0 / 7.0)
    p = p * t2 + jnp.float32(1.0 / 5.0)
    p = p * t2 + jnp.float32(1.0 / 3.0)
    p = p * t2 + 1.0
    return ef * jnp.float32(_LN2) + 2.0 * t * p


def _bf16_pair(w):
    lo = lax.bitcast_convert_type(lax.shift_left(w, 16), jnp.float32)
    hi = lax.bitcast_convert_type(w & jnp.int32(-65536), jnp.float32)
    return lo, hi


def _make_sc_kernel(B, interpret=False):
    bpw = B // NW

    def body(cw_hbm, xw_hbm, co_hbm, w_hbm, cpack_hbm, xpack_hbm,
             cbias_hbm, xbias_hbm, out_hbm,
             idx_cv, idx_xv, idx4_cv, idx4_xv, rows_c, rows_x, co_v, w_v,
             bc_v, bx_v, part_v, sem, semb):
        wid = lax.axis_index("c") * NS + lax.axis_index("s")
        base = wid * bpw
        pltpu.sync_copy(cw_hbm.at[pl.ds(base, bpw)], idx_cv)
        pltpu.sync_copy(xw_hbm.at[pl.ds(base, bpw)], idx_xv)
        pltpu.sync_copy(co_hbm.at[pl.ds(base, bpw)], co_v)
        pltpu.sync_copy(w_hbm.at[pl.ds(base, bpw)], w_v)
        cpb1 = pltpu.async_copy(cbias_hbm.at[idx_cv], bc_v, semb)
        cpb2 = pltpu.async_copy(xbias_hbm.at[idx_xv], bx_v, semb)

        for g in range(bpw // L):
            sl = pl.ds(g * L, L)
            idx4_cv[sl] = lax.shift_right_logical(idx_cv[sl], 2)
            idx4_xv[sl] = lax.shift_right_logical(idx_xv[sl], 2)
        cp1 = pltpu.async_copy(cpack_hbm.at[idx4_cv], rows_c, sem)
        cp2 = pltpu.async_copy(xpack_hbm.at[idx4_xv], rows_x, sem)
        cp1.wait()
        cp2.wait()
        cpb1.wait()
        cpb2.wait()

        zero = jnp.zeros((L,), jnp.float32)
        lane = lax.iota(jnp.int32, L)
        s_wa2 = zero
        s_wa = zero
        s_w = zero
        s_b = zero
        s_b2 = zero
        for g in range(bpw // L):
            sl = pl.ds(g * L, L)
            row = g * L + lane
            colc = (idx_cv[sl] & 3) * _WORDS
            colx = (idx_xv[sl] & 3) * _WORDS

            def dot_step(t, acc, row=row, colc=colc, colx=colx):
                wc = plsc.load_gather(rows_c, [row, colc + t])
                wx = plsc.load_gather(rows_x, [row, colx + t])
                c0, c1 = _bf16_pair(wc)
                x0, x1 = _bf16_pair(wx)
                return acc + (c0 * x0 + c1 * x1)

            sim = lax.fori_loop(0, _WORDS, dot_step, zero)
            a = sim - _vlog(co_v[sl])
            wg = w_v[sl]
            s_wa2 = s_wa2 + wg * a * a
            s_wa = s_wa + wg * a
            s_w = s_w + wg
            bg = bc_v[sl] + bx_v[sl]
            s_b = s_b + bg
            s_b2 = s_b2 + bg * bg

        part_v[pl.ds(0 * L, L)] = s_wa2
        part_v[pl.ds(1 * L, L)] = s_wa
        part_v[pl.ds(2 * L, L)] = s_w
        part_v[pl.ds(3 * L, L)] = s_b
        part_v[pl.ds(4 * L, L)] = s_b2
        pltpu.sync_copy(part_v, out_hbm.at[pl.ds(wid * _NSTAT * L, _NSTAT * L)])

    return pl.kernel(
        body,
        out_type=jax.ShapeDtypeStruct((NW * _NSTAT * L,), jnp.float32),
        mesh=plsc.VectorSubcoreMesh(core_axis_name="c", subcore_axis_name="s",
                                    num_cores=NC),
        scratch_types=[
            pltpu.VMEM((bpw,), jnp.int32),
            pltpu.VMEM((bpw,), jnp.int32),
            pltpu.VMEM((bpw,), jnp.int32),
            pltpu.VMEM((bpw,), jnp.int32),
            pltpu.VMEM((bpw, _PACK * _WORDS), jnp.int32),
            pltpu.VMEM((bpw, _PACK * _WORDS), jnp.int32),
            pltpu.VMEM((bpw,), jnp.float32),
            pltpu.VMEM((bpw,), jnp.float32),
            pltpu.VMEM((bpw,), jnp.float32),
            pltpu.VMEM((bpw,), jnp.float32),
            pltpu.VMEM((_NSTAT * L,), jnp.float32),
            pltpu.SemaphoreType.DMA,
            pltpu.SemaphoreType.DMA,
        ],
        compiler_params=pltpu.CompilerParams(needs_layout_passes=False),
        interpret=interpret,
    )


def _pack_table(emb):
    V, D = emb.shape
    b = emb.astype(jnp.bfloat16).reshape(V // _PACK, _PACK * D // 2, 2)
    return lax.bitcast_convert_type(b, jnp.int32)


def kernel(center_word, context_word, co_mat_val, weight_mat_val,
           center_embedding, context_embedding, center_bias, context_bias):
    B = center_word.shape[0]
    V, D = center_embedding.shape
    cw = center_word.astype(jnp.int32)
    xw = context_word.astype(jnp.int32)
    co = co_mat_val.astype(jnp.float32)
    wv = weight_mat_val.astype(jnp.float32)
    cb = lax.squeeze(center_bias.astype(jnp.float32), dimensions=(1,))
    xb = lax.squeeze(context_bias.astype(jnp.float32), dimensions=(1,))

    partials = _make_sc_kernel(B)(
        cw, xw, co, wv, _pack_table(center_embedding),
        _pack_table(context_embedding), cb, xb)
    p = partials.reshape(NW, _NSTAT, L).sum(axis=(0, 2))
    s_wa2, s_wa, s_w, s_b, s_b2 = p[0], p[1], p[2], p[3], p[4]
    return 0.5 * (B * s_wa2 + 2.0 * s_b * s_wa + s_b2 * s_w)


# speedup vs baseline: 12.0774x; 12.0774x over previous
import functools

import jax
import jax.numpy as jnp
from jax import lax
from jax.experimental import pallas as pl
from jax.experimental.pallas import tpu as pltpu
from jax.experimental.pallas import tpu_sc as plsc

NC = 2
NS = 16
L = 16
NW = NC * NS
_NSTAT = 5
_PACK = 4
_WORDS = 32

_SQRT2 = 1.4142135623730951
_LN2 = 0.6931471805599453


def _vlog(x):
    bits = lax.bitcast_convert_type(x, jnp.int32)
    e = lax.shift_right_logical(bits, 23) - 127
    m_bits = (bits & jnp.int32(0x7FFFFF)) | jnp.int32(0x3F800000)
    m = lax.bitcast_convert_type(m_bits, jnp.float32)
    big = m > _SQRT2
    m = jnp.where(big, 0.5 * m, m)
    e = e + jnp.where(big, 1, 0)
    ef = e.astype(jnp.float32)
    t = (m - 1.0) / (m + 1.0)
    t2 = t * t
    p = jnp.float32(1.0 / 7.0)
    p = p * t2 + jnp.float32(1.0 / 5.0)
    p = p * t2 + jnp.float32(1.0 / 3.0)
    p = p * t2 + 1.0
    return ef * jnp.float32(_LN2) + 2.0 * t * p


def _bf16_pair(w):
    lo = lax.bitcast_convert_type(lax.shift_left(w, 16), jnp.float32)
    hi = lax.bitcast_convert_type(w & jnp.int32(-65536), jnp.float32)
    return lo, hi


def _make_sc_kernel(B, interpret=False):
    bpw = B // NW

    def body(cw_hbm, xw_hbm, co_hbm, w_hbm, cpack_hbm, xpack_hbm,
             cbias_hbm, xbias_hbm, out_hbm,
             idx_cv, idx_xv, idx4_cv, idx4_xv, rows_c, rows_x, co_v, w_v,
             bc_v, bx_v, part_v, sem, semb):
        wid = lax.axis_index("c") * NS + lax.axis_index("s")
        base = wid * bpw
        pltpu.sync_copy(cw_hbm.at[pl.ds(base, bpw)], idx_cv)
        pltpu.sync_copy(xw_hbm.at[pl.ds(base, bpw)], idx_xv)
        pltpu.sync_copy(co_hbm.at[pl.ds(base, bpw)], co_v)
        pltpu.sync_copy(w_hbm.at[pl.ds(base, bpw)], w_v)
        cpb1 = pltpu.async_copy(cbias_hbm.at[idx_cv], bc_v, semb)
        cpb2 = pltpu.async_copy(xbias_hbm.at[idx_xv], bx_v, semb)

        for g in range(bpw // L):
            sl = pl.ds(g * L, L)
            idx4_cv[sl] = lax.shift_right_logical(idx_cv[sl], 2)
            idx4_xv[sl] = lax.shift_right_logical(idx_xv[sl], 2)
        cp1 = pltpu.async_copy(cpack_hbm.at[idx4_cv], rows_c, sem)
        cp2 = pltpu.async_copy(xpack_hbm.at[idx4_xv], rows_x, sem)
        cp1.wait()
        cp2.wait()
        cpb1.wait()
        cpb2.wait()

        zero = jnp.zeros((L,), jnp.float32)
        lane = lax.iota(jnp.int32, L)
        s_wa2 = zero
        s_wa = zero
        s_w = zero
        s_b = zero
        s_b2 = zero
        for g in range(bpw // L):
            sl = pl.ds(g * L, L)
            row = g * L + lane
            colc = (idx_cv[sl] & 3) * _WORDS
            colx = (idx_xv[sl] & 3) * _WORDS

            def dot_step(t, acc, row=row, colc=colc, colx=colx):
                wc = plsc.load_gather(rows_c, [row, colc + t])
                wx = plsc.load_gather(rows_x, [row, colx + t])
                c0, c1 = _bf16_pair(wc)
                x0, x1 = _bf16_pair(wx)
                return acc + (c0 * x0 + c1 * x1)

            sim = lax.fori_loop(0, _WORDS, dot_step, zero)
            a = sim - _vlog(co_v[sl])
            wg = w_v[sl]
            s_wa2 = s_wa2 + wg * a * a
            s_wa = s_wa + wg * a
            s_w = s_w + wg
            bg = bc_v[sl] + bx_v[sl]
            s_b = s_b + bg
            s_b2 = s_b2 + bg * bg

        part_v[pl.ds(0 * L, L)] = s_wa2
        part_v[pl.ds(1 * L, L)] = s_wa
        part_v[pl.ds(2 * L, L)] = s_w
        part_v[pl.ds(3 * L, L)] = s_b
        part_v[pl.ds(4 * L, L)] = s_b2
        pltpu.sync_copy(part_v, out_hbm.at[pl.ds(wid * _NSTAT * L, _NSTAT * L)])

    return pl.kernel(
        body,
        out_type=jax.ShapeDtypeStruct((NW * _NSTAT * L,), jnp.float32),
        mesh=plsc.VectorSubcoreMesh(core_axis_name="c", subcore_axis_name="s",
                                    num_cores=NC),
        scratch_types=[
            pltpu.VMEM((bpw,), jnp.int32),
            pltpu.VMEM((bpw,), jnp.int32),
            pltpu.VMEM((bpw,), jnp.int32),
            pltpu.VMEM((bpw,), jnp.int32),
            pltpu.VMEM((bpw, _PACK * _WORDS), jnp.int32),
            pltpu.VMEM((bpw, _PACK * _WORDS), jnp.int32),
            pltpu.VMEM((bpw,), jnp.float32),
            pltpu.VMEM((bpw,), jnp.float32),
            pltpu.VMEM((bpw,), jnp.float32),
            pltpu.VMEM((bpw,), jnp.float32),
            pltpu.VMEM((_NSTAT * L,), jnp.float32),
            pltpu.SemaphoreType.DMA,
            pltpu.SemaphoreType.DMA,
        ],
        compiler_params=pltpu.CompilerParams(needs_layout_passes=False),
        interpret=interpret,
    )


def _pack_table(emb):
    V, D = emb.shape
    b16 = lax.bitcast_convert_type(emb.astype(jnp.bfloat16), jnp.uint16)
    lo = b16[:, 0::2].astype(jnp.int32)
    hi = b16[:, 1::2].astype(jnp.int32)
    w = lo | lax.shift_left(hi, 16)
    return w.reshape(V // _PACK, _PACK * D // 2)


def kernel(center_word, context_word, co_mat_val, weight_mat_val,
           center_embedding, context_embedding, center_bias, context_bias):
    B = center_word.shape[0]
    V, D = center_embedding.shape
    cw = center_word.astype(jnp.int32)
    xw = context_word.astype(jnp.int32)
    co = co_mat_val.astype(jnp.float32)
    wv = weight_mat_val.astype(jnp.float32)
    cb = lax.squeeze(center_bias.astype(jnp.float32), dimensions=(1,))
    xb = lax.squeeze(context_bias.astype(jnp.float32), dimensions=(1,))

    partials = _make_sc_kernel(B)(
        cw, xw, co, wv, _pack_table(center_embedding),
        _pack_table(context_embedding), cb, xb)
    p = partials.reshape(NW, _NSTAT, L).sum(axis=(0, 2))
    s_wa2, s_wa, s_w, s_b, s_b2 = p[0], p[1], p[2], p[3], p[4]
    return 0.5 * (B * s_wa2 + 2.0 * s_b * s_wa + s_b2 * s_w)


# speedup vs baseline: 39.4629x; 3.2675x over previous
import functools

import jax
import jax.numpy as jnp
from jax import lax
from jax.experimental import pallas as pl
from jax.experimental.pallas import tpu as pltpu
from jax.experimental.pallas import tpu_sc as plsc

NC = 2
NS = 16
L = 16
NW = NC * NS
_NSTAT = 5
_PACK = 2
_WORDS = 64

_SQRT2 = 1.4142135623730951
_LN2 = 0.6931471805599453


def _vlog(x):
    bits = lax.bitcast_convert_type(x, jnp.int32)
    e = lax.shift_right_logical(bits, 23) - 127
    m_bits = (bits & jnp.int32(0x7FFFFF)) | jnp.int32(0x3F800000)
    m = lax.bitcast_convert_type(m_bits, jnp.float32)
    big = m > _SQRT2
    m = jnp.where(big, 0.5 * m, m)
    e = e + jnp.where(big, 1, 0)
    ef = e.astype(jnp.float32)
    t = (m - 1.0) / (m + 1.0)
    t2 = t * t
    p = jnp.float32(1.0 / 7.0)
    p = p * t2 + jnp.float32(1.0 / 5.0)
    p = p * t2 + jnp.float32(1.0 / 3.0)
    p = p * t2 + 1.0
    return ef * jnp.float32(_LN2) + 2.0 * t * p


def _bf16_pair(w):
    lo = lax.bitcast_convert_type(lax.shift_left(w, 16), jnp.float32)
    hi = lax.bitcast_convert_type(w & jnp.int32(-65536), jnp.float32)
    return lo, hi


def _make_sc_kernel(B, interpret=False):
    bpw = B // NW

    def body(cw_hbm, xw_hbm, co_hbm, w_hbm, cpack_hbm, xpack_hbm,
             cbias_hbm, xbias_hbm, out_hbm,
             idx_cv, idx_xv, idx4_cv, idx4_xv, rows_c, rows_x, co_v, w_v,
             bc_v, bx_v, part_v, sem, semb):
        wid = lax.axis_index("c") * NS + lax.axis_index("s")
        base = wid * bpw
        pltpu.sync_copy(cw_hbm.at[pl.ds(base, bpw)], idx_cv)
        pltpu.sync_copy(xw_hbm.at[pl.ds(base, bpw)], idx_xv)
        pltpu.sync_copy(co_hbm.at[pl.ds(base, bpw)], co_v)
        pltpu.sync_copy(w_hbm.at[pl.ds(base, bpw)], w_v)
        cpb1 = pltpu.async_copy(cbias_hbm.at[idx_cv], bc_v, semb)
        cpb2 = pltpu.async_copy(xbias_hbm.at[idx_xv], bx_v, semb)

        for g in range(bpw // L):
            sl = pl.ds(g * L, L)
            idx4_cv[sl] = lax.shift_right_logical(idx_cv[sl], 1)
            idx4_xv[sl] = lax.shift_right_logical(idx_xv[sl], 1)
        cp1 = pltpu.async_copy(cpack_hbm.at[idx4_cv], rows_c, sem)
        cp2 = pltpu.async_copy(xpack_hbm.at[idx4_xv], rows_x, sem)
        cp1.wait()
        cp2.wait()
        cpb1.wait()
        cpb2.wait()

        zero = jnp.zeros((L,), jnp.float32)
        lane = lax.iota(jnp.int32, L)
        s_wa2 = zero
        s_wa = zero
        s_w = zero
        s_b = zero
        s_b2 = zero
        for g in range(bpw // L):
            sl = pl.ds(g * L, L)
            row = g * L + lane
            colc = (idx_cv[sl] & 1) * _WORDS
            colx = (idx_xv[sl] & 1) * _WORDS

            def dot_step(t, acc, row=row, colc=colc, colx=colx):
                wc = plsc.load_gather(rows_c, [row, colc + t])
                wx = plsc.load_gather(rows_x, [row, colx + t])
                return acc + wc * wx

            sim = lax.fori_loop(0, _WORDS, dot_step, zero)
            a = sim - _vlog(co_v[sl])
            wg = w_v[sl]
            s_wa2 = s_wa2 + wg * a * a
            s_wa = s_wa + wg * a
            s_w = s_w + wg
            bg = bc_v[sl] + bx_v[sl]
            s_b = s_b + bg
            s_b2 = s_b2 + bg * bg

        part_v[pl.ds(0 * L, L)] = s_wa2
        part_v[pl.ds(1 * L, L)] = s_wa
        part_v[pl.ds(2 * L, L)] = s_w
        part_v[pl.ds(3 * L, L)] = s_b
        part_v[pl.ds(4 * L, L)] = s_b2
        pltpu.sync_copy(part_v, out_hbm.at[pl.ds(wid * _NSTAT * L, _NSTAT * L)])

    return pl.kernel(
        body,
        out_type=jax.ShapeDtypeStruct((NW * _NSTAT * L,), jnp.float32),
        mesh=plsc.VectorSubcoreMesh(core_axis_name="c", subcore_axis_name="s",
                                    num_cores=NC),
        scratch_types=[
            pltpu.VMEM((bpw,), jnp.int32),
            pltpu.VMEM((bpw,), jnp.int32),
            pltpu.VMEM((bpw,), jnp.int32),
            pltpu.VMEM((bpw,), jnp.int32),
            pltpu.VMEM((bpw, _PACK * _WORDS), jnp.float32),
            pltpu.VMEM((bpw, _PACK * _WORDS), jnp.float32),
            pltpu.VMEM((bpw,), jnp.float32),
            pltpu.VMEM((bpw,), jnp.float32),
            pltpu.VMEM((bpw,), jnp.float32),
            pltpu.VMEM((bpw,), jnp.float32),
            pltpu.VMEM((_NSTAT * L,), jnp.float32),
            pltpu.SemaphoreType.DMA,
            pltpu.SemaphoreType.DMA,
        ],
        compiler_params=pltpu.CompilerParams(needs_layout_passes=False),
        interpret=interpret,
    )


def _pack_table(emb):
    V, D = emb.shape
    return emb.reshape(V // _PACK, _PACK * D)


def kernel(center_word, context_word, co_mat_val, weight_mat_val,
           center_embedding, context_embedding, center_bias, context_bias):
    B = center_word.shape[0]
    V, D = center_embedding.shape
    cw = center_word.astype(jnp.int32)
    xw = context_word.astype(jnp.int32)
    co = co_mat_val.astype(jnp.float32)
    wv = weight_mat_val.astype(jnp.float32)
    cb = lax.squeeze(center_bias.astype(jnp.float32), dimensions=(1,))
    xb = lax.squeeze(context_bias.astype(jnp.float32), dimensions=(1,))

    partials = _make_sc_kernel(B)(
        cw, xw, co, wv, _pack_table(center_embedding),
        _pack_table(context_embedding), cb, xb)
    p = partials.reshape(NW, _NSTAT, L).sum(axis=(0, 2))
    s_wa2, s_wa, s_w, s_b, s_b2 = p[0], p[1], p[2], p[3], p[4]
    return 0.5 * (B * s_wa2 + 2.0 * s_b * s_wa + s_b2 * s_w)


# speedup vs baseline: 196.3498x; 4.9756x over previous
import functools

import jax
import jax.numpy as jnp
from jax import lax
from jax.experimental import pallas as pl
from jax.experimental.pallas import tpu as pltpu
from jax.experimental.pallas import tpu_sc as plsc

NC = 2
NS = 16
L = 16
NW = NC * NS
_NSTAT = 5

_SQRT2 = 1.4142135623730951
_LN2 = 0.6931471805599453


def _vlog(x):
    bits = lax.bitcast_convert_type(x, jnp.int32)
    e = lax.shift_right_logical(bits, 23) - 127
    m_bits = (bits & jnp.int32(0x7FFFFF)) | jnp.int32(0x3F800000)
    m = lax.bitcast_convert_type(m_bits, jnp.float32)
    big = m > _SQRT2
    m = jnp.where(big, 0.5 * m, m)
    e = e + jnp.where(big, 1, 0)
    ef = e.astype(jnp.float32)
    t = (m - 1.0) / (m + 1.0)
    t2 = t * t
    p = jnp.float32(1.0 / 7.0)
    p = p * t2 + jnp.float32(1.0 / 5.0)
    p = p * t2 + jnp.float32(1.0 / 3.0)
    p = p * t2 + 1.0
    return ef * jnp.float32(_LN2) + 2.0 * t * p


def _make_sc_kernel(B, D, interpret=False):
    bpw = B // NW

    def body(cw_hbm, xw_hbm, co_hbm, w_hbm, cembT_hbm, xembT_hbm,
             cbiasT_hbm, xbiasT_hbm, out_hbm,
             idx_cv, idx_xv, blk_c, blk_x, rowsT_c, rowsT_x, co_v, w_v,
             bbl_c, bbl_x, bcT_v, bxT_v, part_v, sem, semb):
        wid = lax.axis_index("c") * NS + lax.axis_index("s")
        base = wid * bpw
        pltpu.sync_copy(cw_hbm.at[pl.ds(base, bpw)], idx_cv)
        pltpu.sync_copy(xw_hbm.at[pl.ds(base, bpw)], idx_xv)
        pltpu.sync_copy(co_hbm.at[pl.ds(base, bpw)], co_v)
        pltpu.sync_copy(w_hbm.at[pl.ds(base, bpw)], w_v)

        zero = jnp.zeros((L,), jnp.float32)
        lane = lax.iota(jnp.int32, L)
        zeros16 = jnp.zeros((L,), jnp.int32)
        lane0 = lane == 0

        def fetch_group(g, _):
            gb = g * L
            vecc = idx_cv[pl.ds(gb, L)]
            vecx = idx_xv[pl.ds(gb, L)]
            for k in range(L):
                vc = vecc[k]
                vx = vecx[k]
                j = gb + k
                bc = pl.multiple_of(vc & jnp.int32(-128), 128)
                bx = pl.multiple_of(vx & jnp.int32(-128), 128)
                cp1 = pltpu.async_copy(cembT_hbm.at[:, pl.ds(bc, 128)],
                                       blk_c, sem)
                cp2 = pltpu.async_copy(xembT_hbm.at[:, pl.ds(bx, 128)],
                                       blk_x, sem)
                cp3 = pltpu.async_copy(cbiasT_hbm.at[:, pl.ds(bc, 128)],
                                       bbl_c, semb)
                cp4 = pltpu.async_copy(xbiasT_hbm.at[:, pl.ds(bx, 128)],
                                       bbl_x, semb)
                cp1.wait()
                cp2.wait()
                cp3.wait()
                cp4.wait()
                colc = jnp.full((L,), vc & 127, jnp.int32)
                colx = jnp.full((L,), vx & 127, jnp.int32)
                jcol = jnp.full((L,), j, jnp.int32)
                for db in range(D // L):
                    drow = db * L + lane
                    plsc.store_scatter(rowsT_c, [drow, jcol],
                                       plsc.load_gather(blk_c, [drow, colc]))
                    plsc.store_scatter(rowsT_x, [drow, jcol],
                                       plsc.load_gather(blk_x, [drow, colx]))
                plsc.store_scatter(bcT_v, [zeros16, jcol],
                                   plsc.load_gather(bbl_c, [zeros16, colc]),
                                   mask=lane0)
                plsc.store_scatter(bxT_v, [zeros16, jcol],
                                   plsc.load_gather(bbl_x, [zeros16, colx]),
                                   mask=lane0)
            return 0

        lax.fori_loop(0, bpw // L, fetch_group, 0)
        s_wa2 = zero
        s_wa = zero
        s_w = zero
        s_b = zero
        s_b2 = zero
        for g in range(bpw // L):
            sl = pl.ds(g * L, L)
            col = g * L + lane

            def dot_step(dd, acc, col=col):
                drow = jnp.full((L,), dd, jnp.int32)
                av = plsc.load_gather(rowsT_c, [drow, col])
                bv = plsc.load_gather(rowsT_x, [drow, col])
                return acc + av * bv

            sim = lax.fori_loop(0, D, dot_step, zero)
            a = sim - _vlog(co_v[sl])
            wg = w_v[sl]
            s_wa2 = s_wa2 + wg * a * a
            s_wa = s_wa + wg * a
            s_w = s_w + wg
            bg = (plsc.load_gather(bcT_v, [zeros16, col])
                  + plsc.load_gather(bxT_v, [zeros16, col]))
            s_b = s_b + bg
            s_b2 = s_b2 + bg * bg

        part_v[pl.ds(0 * L, L)] = s_wa2
        part_v[pl.ds(1 * L, L)] = s_wa
        part_v[pl.ds(2 * L, L)] = s_w
        part_v[pl.ds(3 * L, L)] = s_b
        part_v[pl.ds(4 * L, L)] = s_b2
        pltpu.sync_copy(part_v, out_hbm.at[pl.ds(wid * _NSTAT * L, _NSTAT * L)])

    return pl.kernel(
        body,
        out_type=jax.ShapeDtypeStruct((NW * _NSTAT * L,), jnp.float32),
        mesh=plsc.VectorSubcoreMesh(core_axis_name="c", subcore_axis_name="s",
                                    num_cores=NC),
        scratch_types=[
            pltpu.VMEM((bpw,), jnp.int32),
            pltpu.VMEM((bpw,), jnp.int32),
            pltpu.VMEM((D, 128), jnp.float32),
            pltpu.VMEM((D, 128), jnp.float32),
            pltpu.VMEM((D, bpw), jnp.float32),
            pltpu.VMEM((D, bpw), jnp.float32),
            pltpu.VMEM((bpw,), jnp.float32),
            pltpu.VMEM((bpw,), jnp.float32),
            pltpu.VMEM((1, 128), jnp.float32),
            pltpu.VMEM((1, 128), jnp.float32),
            pltpu.VMEM((1, bpw), jnp.float32),
            pltpu.VMEM((1, bpw), jnp.float32),
            pltpu.VMEM((_NSTAT * L,), jnp.float32),
            pltpu.SemaphoreType.DMA,
            pltpu.SemaphoreType.DMA,
        ],
        compiler_params=pltpu.CompilerParams(needs_layout_passes=False),
        interpret=interpret,
    )


def kernel(center_word, context_word, co_mat_val, weight_mat_val,
           center_embedding, context_embedding, center_bias, context_bias):
    B = center_word.shape[0]
    V, D = center_embedding.shape
    cw = center_word.astype(jnp.int32)
    xw = context_word.astype(jnp.int32)
    co = co_mat_val.astype(jnp.float32)
    wv = weight_mat_val.astype(jnp.float32)

    partials = _make_sc_kernel(B, D)(
        cw, xw, co, wv,
        center_embedding.T, context_embedding.T,
        center_bias.astype(jnp.float32).T, context_bias.astype(jnp.float32).T)
    p = partials.reshape(NW, _NSTAT, L).sum(axis=(0, 2))
    s_wa2, s_wa, s_w, s_b, s_b2 = p[0], p[1], p[2], p[3], p[4]
    return 0.5 * (B * s_wa2 + 2.0 * s_b * s_wa + s_b2 * s_w)


# speedup vs baseline: 312.3422x; 1.5907x over previous
import functools

import jax
import jax.numpy as jnp
from jax import lax
from jax.experimental import pallas as pl
from jax.experimental.pallas import tpu as pltpu
from jax.experimental.pallas import tpu_sc as plsc

NC = 2
NS = 16
L = 16
NW = NC * NS
_NSTAT = 5

_SQRT2 = 1.4142135623730951
_LN2 = 0.6931471805599453


def _vlog(x):
    bits = lax.bitcast_convert_type(x, jnp.int32)
    e = lax.shift_right_logical(bits, 23) - 127
    m_bits = (bits & jnp.int32(0x7FFFFF)) | jnp.int32(0x3F800000)
    m = lax.bitcast_convert_type(m_bits, jnp.float32)
    big = m > _SQRT2
    m = jnp.where(big, 0.5 * m, m)
    e = e + jnp.where(big, 1, 0)
    ef = e.astype(jnp.float32)
    t = (m - 1.0) / (m + 1.0)
    t2 = t * t
    p = jnp.float32(1.0 / 7.0)
    p = p * t2 + jnp.float32(1.0 / 5.0)
    p = p * t2 + jnp.float32(1.0 / 3.0)
    p = p * t2 + 1.0
    return ef * jnp.float32(_LN2) + 2.0 * t * p


def _make_sc_kernel(B, D, interpret=False):
    bpw = B // NW

    def body(cw_hbm, xw_hbm, co_hbm, w_hbm, cembT_hbm, xembT_hbm,
             cbiasT_hbm, xbiasT_hbm, out_hbm,
             idx_cv, idx_xv, blk_c0, blk_c1, blk_c2, blk_c3,
             blk_x0, blk_x1, blk_x2, blk_x3, rowsT_c, rowsT_x, co_v, w_v,
             bbl_c0, bbl_c1, bbl_c2, bbl_c3, bbl_x0, bbl_x1, bbl_x2, bbl_x3,
             bcT_v, bxT_v, part_v, sem0, sem1, sem2, sem3):
        wid = lax.axis_index("c") * NS + lax.axis_index("s")
        base = wid * bpw
        pltpu.sync_copy(cw_hbm.at[pl.ds(base, bpw)], idx_cv)
        pltpu.sync_copy(xw_hbm.at[pl.ds(base, bpw)], idx_xv)
        pltpu.sync_copy(co_hbm.at[pl.ds(base, bpw)], co_v)
        pltpu.sync_copy(w_hbm.at[pl.ds(base, bpw)], w_v)

        zero = jnp.zeros((L,), jnp.float32)
        lane = lax.iota(jnp.int32, L)
        zeros16 = jnp.zeros((L,), jnp.int32)
        lane0 = lane == 0

        NSLOT = 4
        NPRE = 3
        blks_c = [blk_c0, blk_c1, blk_c2, blk_c3]
        blks_x = [blk_x0, blk_x1, blk_x2, blk_x3]
        bbls_c = [bbl_c0, bbl_c1, bbl_c2, bbl_c3]
        bbls_x = [bbl_x0, bbl_x1, bbl_x2, bbl_x3]
        sems = [sem0, sem1, sem2, sem3]

        def fetch_group(g, _):
            gb = g * L
            vecc = idx_cv[pl.ds(gb, L)]
            vecx = idx_xv[pl.ds(gb, L)]

            def fire(k):
                s = k % NSLOT
                vc = vecc[k]
                vx = vecx[k]
                bc = pl.multiple_of(vc & jnp.int32(-128), 128)
                bx = pl.multiple_of(vx & jnp.int32(-128), 128)
                return (
                    pltpu.async_copy(cembT_hbm.at[:, pl.ds(bc, 128)],
                                     blks_c[s], sems[s]),
                    pltpu.async_copy(xembT_hbm.at[:, pl.ds(bx, 128)],
                                     blks_x[s], sems[s]),
                    pltpu.async_copy(cbiasT_hbm.at[:, pl.ds(bc, 128)],
                                     bbls_c[s], sems[s]),
                    pltpu.async_copy(xbiasT_hbm.at[:, pl.ds(bx, 128)],
                                     bbls_x[s], sems[s]),
                )

            descs = {}
            for k in range(NPRE):
                descs[k] = fire(k)
            for k in range(L):
                if k + NPRE < L:
                    descs[k + NPRE] = fire(k + NPRE)
                for cp in descs.pop(k):
                    cp.wait()
                s = k % NSLOT
                vc = vecc[k]
                vx = vecx[k]
                colc = jnp.full((L,), vc & 127, jnp.int32)
                colx = jnp.full((L,), vx & 127, jnp.int32)
                jcol = jnp.full((L,), gb + k, jnp.int32)
                for db in range(D // L):
                    drow = db * L + lane
                    plsc.store_scatter(
                        rowsT_c, [drow, jcol],
                        plsc.load_gather(blks_c[s], [drow, colc]))
                    plsc.store_scatter(
                        rowsT_x, [drow, jcol],
                        plsc.load_gather(blks_x[s], [drow, colx]))
                plsc.store_scatter(
                    bcT_v, [zeros16, jcol],
                    plsc.load_gather(bbls_c[s], [zeros16, colc]), mask=lane0)
                plsc.store_scatter(
                    bxT_v, [zeros16, jcol],
                    plsc.load_gather(bbls_x[s], [zeros16, colx]), mask=lane0)
            return 0

        lax.fori_loop(0, bpw // L, fetch_group, 0)
        s_wa2 = zero
        s_wa = zero
        s_w = zero
        s_b = zero
        s_b2 = zero
        for g in range(bpw // L):
            sl = pl.ds(g * L, L)
            col = g * L + lane

            def dot_step(dd, acc, col=col):
                drow = jnp.full((L,), dd, jnp.int32)
                av = plsc.load_gather(rowsT_c, [drow, col])
                bv = plsc.load_gather(rowsT_x, [drow, col])
                return acc + av * bv

            sim = lax.fori_loop(0, D, dot_step, zero)
            a = sim - _vlog(co_v[sl])
            wg = w_v[sl]
            s_wa2 = s_wa2 + wg * a * a
            s_wa = s_wa + wg * a
            s_w = s_w + wg
            bg = (plsc.load_gather(bcT_v, [zeros16, col])
                  + plsc.load_gather(bxT_v, [zeros16, col]))
            s_b = s_b + bg
            s_b2 = s_b2 + bg * bg

        part_v[pl.ds(0 * L, L)] = s_wa2
        part_v[pl.ds(1 * L, L)] = s_wa
        part_v[pl.ds(2 * L, L)] = s_w
        part_v[pl.ds(3 * L, L)] = s_b
        part_v[pl.ds(4 * L, L)] = s_b2
        pltpu.sync_copy(part_v, out_hbm.at[pl.ds(wid * _NSTAT * L, _NSTAT * L)])

    return pl.kernel(
        body,
        out_type=jax.ShapeDtypeStruct((NW * _NSTAT * L,), jnp.float32),
        mesh=plsc.VectorSubcoreMesh(core_axis_name="c", subcore_axis_name="s",
                                    num_cores=NC),
        scratch_types=[
            pltpu.VMEM((bpw,), jnp.int32),
            pltpu.VMEM((bpw,), jnp.int32),
            pltpu.VMEM((D, 128), jnp.float32),
            pltpu.VMEM((D, 128), jnp.float32),
            pltpu.VMEM((D, 128), jnp.float32),
            pltpu.VMEM((D, 128), jnp.float32),
            pltpu.VMEM((D, 128), jnp.float32),
            pltpu.VMEM((D, 128), jnp.float32),
            pltpu.VMEM((D, 128), jnp.float32),
            pltpu.VMEM((D, 128), jnp.float32),
            pltpu.VMEM((D, bpw), jnp.float32),
            pltpu.VMEM((D, bpw), jnp.float32),
            pltpu.VMEM((bpw,), jnp.float32),
            pltpu.VMEM((bpw,), jnp.float32),
            pltpu.VMEM((1, 128), jnp.float32),
            pltpu.VMEM((1, 128), jnp.float32),
            pltpu.VMEM((1, 128), jnp.float32),
            pltpu.VMEM((1, 128), jnp.float32),
            pltpu.VMEM((1, 128), jnp.float32),
            pltpu.VMEM((1, 128), jnp.float32),
            pltpu.VMEM((1, 128), jnp.float32),
            pltpu.VMEM((1, 128), jnp.float32),
            pltpu.VMEM((1, bpw), jnp.float32),
            pltpu.VMEM((1, bpw), jnp.float32),
            pltpu.VMEM((_NSTAT * L,), jnp.float32),
            pltpu.SemaphoreType.DMA,
            pltpu.SemaphoreType.DMA,
            pltpu.SemaphoreType.DMA,
            pltpu.SemaphoreType.DMA,
        ],
        compiler_params=pltpu.CompilerParams(needs_layout_passes=False),
        interpret=interpret,
    )


def kernel(center_word, context_word, co_mat_val, weight_mat_val,
           center_embedding, context_embedding, center_bias, context_bias):
    B = center_word.shape[0]
    V, D = center_embedding.shape
    cw = center_word.astype(jnp.int32)
    xw = context_word.astype(jnp.int32)
    co = co_mat_val.astype(jnp.float32)
    wv = weight_mat_val.astype(jnp.float32)

    partials = _make_sc_kernel(B, D)(
        cw, xw, co, wv,
        center_embedding.T, context_embedding.T,
        center_bias.astype(jnp.float32).T, context_bias.astype(jnp.float32).T)
    p = partials.reshape(NW, _NSTAT, L).sum(axis=(0, 2))
    s_wa2, s_wa, s_w, s_b, s_b2 = p[0], p[1], p[2], p[3], p[4]
    return 0.5 * (B * s_wa2 + 2.0 * s_b * s_wa + s_b2 * s_w)


# speedup vs baseline: 318.3616x; 1.0193x over previous
import functools

import jax
import jax.numpy as jnp
from jax import lax
from jax.experimental import pallas as pl
from jax.experimental.pallas import tpu as pltpu
from jax.experimental.pallas import tpu_sc as plsc

NC = 2
NS = 16
L = 16
NW = NC * NS
_NSTAT = 5

_SQRT2 = 1.4142135623730951
_LN2 = 0.6931471805599453


def _vlog(x):
    bits = lax.bitcast_convert_type(x, jnp.int32)
    e = lax.shift_right_logical(bits, 23) - 127
    m_bits = (bits & jnp.int32(0x7FFFFF)) | jnp.int32(0x3F800000)
    m = lax.bitcast_convert_type(m_bits, jnp.float32)
    big = m > _SQRT2
    m = jnp.where(big, 0.5 * m, m)
    e = e + jnp.where(big, 1, 0)
    ef = e.astype(jnp.float32)
    t = (m - 1.0) / (m + 1.0)
    t2 = t * t
    p = jnp.float32(1.0 / 7.0)
    p = p * t2 + jnp.float32(1.0 / 5.0)
    p = p * t2 + jnp.float32(1.0 / 3.0)
    p = p * t2 + 1.0
    return ef * jnp.float32(_LN2) + 2.0 * t * p


def _make_sc_kernel(B, D, interpret=False):
    bpw = B // NW

    def body(cw_hbm, xw_hbm, co_hbm, w_hbm, cembT_hbm, xembT_hbm,
             cbiasT_hbm, xbiasT_hbm, out_hbm,
             idx_cv, idx_xv, blk_c0, blk_c1, blk_c2, blk_c3,
             blk_x0, blk_x1, blk_x2, blk_x3, prodT_v, co_v, w_v,
             bbl_c0, bbl_c1, bbl_c2, bbl_c3, bbl_x0, bbl_x1, bbl_x2, bbl_x3,
             bcT_v, bxT_v, part_v, sem0, sem1, sem2, sem3):
        wid = lax.axis_index("c") * NS + lax.axis_index("s")
        base = wid * bpw
        pltpu.sync_copy(cw_hbm.at[pl.ds(base, bpw)], idx_cv)
        pltpu.sync_copy(xw_hbm.at[pl.ds(base, bpw)], idx_xv)
        pltpu.sync_copy(co_hbm.at[pl.ds(base, bpw)], co_v)
        pltpu.sync_copy(w_hbm.at[pl.ds(base, bpw)], w_v)

        zero = jnp.zeros((L,), jnp.float32)
        lane = lax.iota(jnp.int32, L)
        zeros16 = jnp.zeros((L,), jnp.int32)
        lane0 = lane == 0

        NSLOT = 4
        NPRE = 3
        blks_c = [blk_c0, blk_c1, blk_c2, blk_c3]
        blks_x = [blk_x0, blk_x1, blk_x2, blk_x3]
        bbls_c = [bbl_c0, bbl_c1, bbl_c2, bbl_c3]
        bbls_x = [bbl_x0, bbl_x1, bbl_x2, bbl_x3]
        sems = [sem0, sem1, sem2, sem3]

        def fetch_group(g, _):
            gb = g * L
            vecc = idx_cv[pl.ds(gb, L)]
            vecx = idx_xv[pl.ds(gb, L)]

            def fire(k):
                s = k % NSLOT
                vc = vecc[k]
                vx = vecx[k]
                bc = pl.multiple_of(vc & jnp.int32(-128), 128)
                bx = pl.multiple_of(vx & jnp.int32(-128), 128)
                return (
                    pltpu.async_copy(cembT_hbm.at[:, pl.ds(bc, 128)],
                                     blks_c[s], sems[s]),
                    pltpu.async_copy(xembT_hbm.at[:, pl.ds(bx, 128)],
                                     blks_x[s], sems[s]),
                    pltpu.async_copy(cbiasT_hbm.at[:, pl.ds(bc, 128)],
                                     bbls_c[s], sems[s]),
                    pltpu.async_copy(xbiasT_hbm.at[:, pl.ds(bx, 128)],
                                     bbls_x[s], sems[s]),
                )

            descs = {}
            for k in range(NPRE):
                descs[k] = fire(k)
            for k in range(L):
                if k + NPRE < L:
                    descs[k + NPRE] = fire(k + NPRE)
                for cp in descs.pop(k):
                    cp.wait()
                s = k % NSLOT
                vc = vecc[k]
                vx = vecx[k]
                colc = jnp.full((L,), vc & 127, jnp.int32)
                colx = jnp.full((L,), vx & 127, jnp.int32)
                jcol = jnp.full((L,), gb + k, jnp.int32)
                prod = zero
                for db in range(D // L):
                    drow = db * L + lane
                    prod = prod + (
                        plsc.load_gather(blks_c[s], [drow, colc])
                        * plsc.load_gather(blks_x[s], [drow, colx]))
                plsc.store_scatter(prodT_v, [lane, jcol], prod)
                plsc.store_scatter(
                    bcT_v, [zeros16, jcol],
                    plsc.load_gather(bbls_c[s], [zeros16, colc]), mask=lane0)
                plsc.store_scatter(
                    bxT_v, [zeros16, jcol],
                    plsc.load_gather(bbls_x[s], [zeros16, colx]), mask=lane0)
            return 0

        lax.fori_loop(0, bpw // L, fetch_group, 0)
        s_wa2 = zero
        s_wa = zero
        s_w = zero
        s_b = zero
        s_b2 = zero
        for g in range(bpw // L):
            sl = pl.ds(g * L, L)
            col = g * L + lane

            sim = zero
            for r in range(L):
                rrow = jnp.full((L,), r, jnp.int32)
                sim = sim + plsc.load_gather(prodT_v, [rrow, col])
            a = sim - _vlog(co_v[sl])
            wg = w_v[sl]
            s_wa2 = s_wa2 + wg * a * a
            s_wa = s_wa + wg * a
            s_w = s_w + wg
            bg = (plsc.load_gather(bcT_v, [zeros16, col])
                  + plsc.load_gather(bxT_v, [zeros16, col]))
            s_b = s_b + bg
            s_b2 = s_b2 + bg * bg

        part_v[pl.ds(0 * L, L)] = s_wa2
        part_v[pl.ds(1 * L, L)] = s_wa
        part_v[pl.ds(2 * L, L)] = s_w
        part_v[pl.ds(3 * L, L)] = s_b
        part_v[pl.ds(4 * L, L)] = s_b2
        pltpu.sync_copy(part_v, out_hbm.at[pl.ds(wid * _NSTAT * L, _NSTAT * L)])

    return pl.kernel(
        body,
        out_type=jax.ShapeDtypeStruct((NW * _NSTAT * L,), jnp.float32),
        mesh=plsc.VectorSubcoreMesh(core_axis_name="c", subcore_axis_name="s",
                                    num_cores=NC),
        scratch_types=[
            pltpu.VMEM((bpw,), jnp.int32),
            pltpu.VMEM((bpw,), jnp.int32),
            pltpu.VMEM((D, 128), jnp.float32),
            pltpu.VMEM((D, 128), jnp.float32),
            pltpu.VMEM((D, 128), jnp.float32),
            pltpu.VMEM((D, 128), jnp.float32),
            pltpu.VMEM((D, 128), jnp.float32),
            pltpu.VMEM((D, 128), jnp.float32),
            pltpu.VMEM((D, 128), jnp.float32),
            pltpu.VMEM((D, 128), jnp.float32),
            pltpu.VMEM((L, bpw), jnp.float32),
            pltpu.VMEM((bpw,), jnp.float32),
            pltpu.VMEM((bpw,), jnp.float32),
            pltpu.VMEM((1, 128), jnp.float32),
            pltpu.VMEM((1, 128), jnp.float32),
            pltpu.VMEM((1, 128), jnp.float32),
            pltpu.VMEM((1, 128), jnp.float32),
            pltpu.VMEM((1, 128), jnp.float32),
            pltpu.VMEM((1, 128), jnp.float32),
            pltpu.VMEM((1, 128), jnp.float32),
            pltpu.VMEM((1, 128), jnp.float32),
            pltpu.VMEM((1, bpw), jnp.float32),
            pltpu.VMEM((1, bpw), jnp.float32),
            pltpu.VMEM((_NSTAT * L,), jnp.float32),
            pltpu.SemaphoreType.DMA,
            pltpu.SemaphoreType.DMA,
            pltpu.SemaphoreType.DMA,
            pltpu.SemaphoreType.DMA,
        ],
        compiler_params=pltpu.CompilerParams(needs_layout_passes=False),
        interpret=interpret,
    )


def kernel(center_word, context_word, co_mat_val, weight_mat_val,
           center_embedding, context_embedding, center_bias, context_bias):
    B = center_word.shape[0]
    V, D = center_embedding.shape
    cw = center_word.astype(jnp.int32)
    xw = context_word.astype(jnp.int32)
    co = co_mat_val.astype(jnp.float32)
    wv = weight_mat_val.astype(jnp.float32)

    partials = _make_sc_kernel(B, D)(
        cw, xw, co, wv,
        center_embedding.T, context_embedding.T,
        center_bias.astype(jnp.float32).T, context_bias.astype(jnp.float32).T)
    p = partials.reshape(NW, _NSTAT, L).sum(axis=(0, 2))
    s_wa2, s_wa, s_w, s_b, s_b2 = p[0], p[1], p[2], p[3], p[4]
    return 0.5 * (B * s_wa2 + 2.0 * s_b * s_wa + s_b2 * s_w)
